# baseline (speedup 1.0000x reference)
import jax
import jax.numpy as jnp
from jax import lax
from jax.experimental import pallas as pl
from jax.experimental.pallas import tpu as pltpu
from jax.experimental.pallas import tpu_sc as plsc

B = 8
H = W = 112
C = 96
N = H * W * C
M = 4 * N
NC = 2
NS = 16
LANES = 16
NCHUNK = 3
CHUNK = M // NCHUNK
NPASS = B * NCHUNK
NT = N // NS
P = 6272
NPIECE = NT // P
NTZ = CHUNK // NS
NZP = NTZ // P
S = 3072
SBUF = S + LANES


def _unpool_body(grad_hbm, arg_hbm, out_hbm, acc, idx_v, val_v, zer_v,
                 cidx_v, cval_v):
    cid = lax.axis_index("c")
    sid = lax.axis_index("s")

    zeros16 = jnp.zeros((LANES,), jnp.float32)

    def fill_zer(i, _):
        zer_v[pl.ds(i * LANES, LANES)] = zeros16
        return _

    lax.fori_loop(0, P // LANES, fill_zer, None)

    def clear_cbuf(i, _):
        cidx_v[pl.ds(i * LANES, LANES)] = jnp.zeros((LANES,), jnp.int32)
        cval_v[pl.ds(i * LANES, LANES)] = zeros16
        return _

    lax.fori_loop(0, SBUF // LANES, clear_cbuf, None)

    def zero_cval(i, _):
        cval_v[pl.ds(i * LANES, LANES)] = zeros16
        return _

    def pass_body(k, _):
        t = 2 * k + cid
        b = t // NCHUNK
        ch = t % NCHUNK
        lo = ch * CHUNK

        def zero_body(j, _):
            pltpu.sync_copy(zer_v, acc.at[pl.ds(sid * NTZ + j * P, P)])
            return _

        lax.fori_loop(0, NZP, zero_body, None)
        plsc.subcore_barrier()

        def piece_body(p, cnt):
            base = b * N + sid * NT + p * P
            pltpu.sync_copy(arg_hbm.at[pl.ds(base, P)], idx_v)
            pltpu.sync_copy(grad_hbm.at[pl.ds(base, P)], val_v)

            def remap_body(i, cnt):
                sl = pl.ds(i * LANES, LANES)
                iv = idx_v[sl]
                vv = val_v[sl]
                m = (iv >= lo) & (iv < lo + CHUNK)
                win = pl.ds(cnt, LANES)
                plsc.store_compressed(cidx_v.at[win], iv - lo, mask=m)
                plsc.store_compressed(cval_v.at[win], vv, mask=m)
                cnt = cnt + plsc.all_reduce_population_count(m)[0]

                def drain():
                    pltpu.sync_copy(cval_v, acc.at[cidx_v], add=True)
                    lax.fori_loop(0, SBUF // LANES, zero_cval, None)

                pl.when(cnt >= S)(drain)
                return jnp.where(cnt >= S, 0, cnt)

            return lax.fori_loop(0, P // LANES, remap_body, cnt)

        cnt = lax.fori_loop(0, NPIECE, piece_body, jnp.int32(0))

        def tail_drain():
            pltpu.sync_copy(cval_v, acc.at[cidx_v], add=True)
            lax.fori_loop(0, SBUF // LANES, zero_cval, None)

        pl.when(cnt > 0)(tail_drain)
        plsc.subcore_barrier()

        pltpu.sync_copy(
            acc.at[pl.ds(sid * NTZ, NTZ)],
            out_hbm.at[pl.ds(b * M + lo + sid * NTZ, NTZ)],
        )
        plsc.subcore_barrier()
        return _

    lax.fori_loop(0, NPASS // NC, pass_body, None)


@jax.jit
def _unpool(grad_flat, arg_flat):
    mesh = plsc.VectorSubcoreMesh(core_axis_name="c", subcore_axis_name="s")
    return pl.kernel(
        _unpool_body,
        out_type=jax.ShapeDtypeStruct((B * M,), jnp.float32),
        mesh=mesh,
        compiler_params=pltpu.CompilerParams(needs_layout_passes=False),
        scratch_types=[
            pltpu.VMEM_SHARED((CHUNK,), jnp.float32),
            pltpu.VMEM((P,), jnp.int32),
            pltpu.VMEM((P,), jnp.float32),
            pltpu.VMEM((P,), jnp.float32),
            pltpu.VMEM((SBUF,), jnp.int32),
            pltpu.VMEM((SBUF,), jnp.float32),
        ],
    )(grad_flat, arg_flat)


def kernel(grad_out, inputs, argmax, batch_size):
    del inputs, batch_size
    grad_flat = grad_out.reshape(B * N)
    arg_flat = argmax.reshape(B * N).astype(jnp.int32)
    out_flat = _unpool(grad_flat, arg_flat)
    return out_flat.reshape(B, 2 * H, 2 * W, C)


# speedup vs baseline: 1.6810x; 1.6810x over previous
import jax
import jax.numpy as jnp
from jax import lax
from jax.experimental import pallas as pl
from jax.experimental.pallas import tpu as pltpu
from jax.experimental.pallas import tpu_sc as plsc

B = 8
H = W = 112
C = 96
N = H * W * C
M = 4 * N
NC = 2
NS = 16
LANES = 16
NCHUNK = 3
CHUNK = M // NCHUNK
NPASS = B * NCHUNK
NT = N // NS
P = 12544
NPIECE = NT // P
NTZ = CHUNK // NS
NZP = NTZ // P


def _unpool_body(grad_hbm, arg_hbm, out_hbm, acc, idx_v, val_v):
    cid = lax.axis_index("c")
    sid = lax.axis_index("s")

    zeros16 = jnp.zeros((LANES,), jnp.float32)

    def pass_body(k, _):
        t = 2 * k + cid
        b = t // NCHUNK
        ch = t % NCHUNK
        lo = ch * CHUNK

        def fill_zeros(i, _):
            val_v[pl.ds(i * LANES, LANES)] = zeros16
            return _

        lax.fori_loop(0, P // LANES, fill_zeros, None)

        def zero_body(j, _):
            pltpu.sync_copy(val_v, acc.at[pl.ds(sid * NTZ + j * P, P)])
            return _

        lax.fori_loop(0, NZP, zero_body, None)
        plsc.subcore_barrier()

        def piece_body(p, _):
            base = b * N + sid * NT + p * P
            pltpu.sync_copy(arg_hbm.at[pl.ds(base, P)], idx_v)
            pltpu.sync_copy(grad_hbm.at[pl.ds(base, P)], val_v)

            def remap_body(i, _):
                sl = pl.ds(i * LANES, LANES)
                iv = idx_v[sl]
                vv = val_v[sl]
                m = (iv >= lo) & (iv < lo + CHUNK)
                idx_v[sl] = jnp.where(m, iv - lo, iv >> 2)
                val_v[sl] = jnp.where(m, vv, 0.0)
                return _

            lax.fori_loop(0, P // LANES, remap_body, None)
            pltpu.sync_copy(val_v, acc.at[idx_v], add=True)
            return _

        lax.fori_loop(0, NPIECE, piece_body, None)
        plsc.subcore_barrier()

        pltpu.sync_copy(
            acc.at[pl.ds(sid * NTZ, NTZ)],
            out_hbm.at[pl.ds(b * M + lo + sid * NTZ, NTZ)],
        )
        plsc.subcore_barrier()
        return _

    lax.fori_loop(0, NPASS // NC, pass_body, None)


@jax.jit
def _unpool(grad_flat, arg_flat):
    mesh = plsc.VectorSubcoreMesh(core_axis_name="c", subcore_axis_name="s")
    return pl.kernel(
        _unpool_body,
        out_type=jax.ShapeDtypeStruct((B * M,), jnp.float32),
        mesh=mesh,
        compiler_params=pltpu.CompilerParams(needs_layout_passes=False),
        scratch_types=[
            pltpu.VMEM_SHARED((CHUNK,), jnp.float32),
            pltpu.VMEM((P,), jnp.int32),
            pltpu.VMEM((P,), jnp.float32),
        ],
    )(grad_flat, arg_flat)


def kernel(grad_out, inputs, argmax, batch_size):
    del inputs, batch_size
    grad_flat = grad_out.reshape(B * N)
    arg_flat = argmax.reshape(B * N).astype(jnp.int32)
    out_flat = _unpool(grad_flat, arg_flat)
    return out_flat.reshape(B, 2 * H, 2 * W, C)


# speedup vs baseline: 2.0398x; 1.2134x over previous
import jax
import jax.numpy as jnp
from jax import lax
from jax.experimental import pallas as pl
from jax.experimental.pallas import tpu as pltpu
from jax.experimental.pallas import tpu_sc as plsc

B = 8
H = W = 112
C = 96
N = H * W * C
M = 4 * N
NC = 2
NS = 16
LANES = 16
NCHUNK = 3
CHUNK = M // NCHUNK
NPASS = B * NCHUNK
NT = N // NS
P = 6272
NPIECE = NT // P
NTZ = CHUNK // NS
NZP = NTZ // P


def _unpool_body(grad_hbm, arg_hbm, out_hbm, acc,
                 idx0, val0, idx1, val1, lsem0, lsem1, ssem0, ssem1):
    cid = lax.axis_index("c")
    sid = lax.axis_index("s")

    idx_s = (idx0, idx1)
    val_s = (val0, val1)
    lsem = (lsem0, lsem1)
    ssem = (ssem0, ssem1)

    zeros16 = jnp.zeros((LANES,), jnp.float32)

    def start_loads(b, p, slot):
        base = b * N + sid * NT + p * P
        pltpu.make_async_copy(
            arg_hbm.at[pl.ds(base, P)], idx_s[slot], lsem[slot]).start()
        pltpu.make_async_copy(
            grad_hbm.at[pl.ds(base, P)], val_s[slot], lsem[slot]).start()

    def wait_loads(b, p, slot):
        base = b * N + sid * NT + p * P
        pltpu.make_async_copy(
            arg_hbm.at[pl.ds(base, P)], idx_s[slot], lsem[slot]).wait()
        pltpu.make_async_copy(
            grad_hbm.at[pl.ds(base, P)], val_s[slot], lsem[slot]).wait()

    def pass_body(k, _):
        t = 2 * k + cid
        b = t // NCHUNK
        ch = t % NCHUNK
        lo = ch * CHUNK

        def fill_zeros(i, _):
            val0[pl.ds(i * LANES, LANES)] = zeros16
            return _

        lax.fori_loop(0, P // LANES, fill_zeros, None)

        def zero_body(j, _):
            pltpu.sync_copy(val0, acc.at[pl.ds(sid * NTZ + j * P, P)])
            return _

        lax.fori_loop(0, NZP, zero_body, None)
        plsc.subcore_barrier()

        start_loads(b, 0, 0)
        for p in range(NPIECE):
            slot = p % 2
            wait_loads(b, p, slot)

            def remap_body(i, _, slot=slot):
                sl = pl.ds(i * LANES, LANES)
                iv = idx_s[slot][sl]
                vv = val_s[slot][sl]
                m = (iv >= lo) & (iv < lo + CHUNK)
                idx_s[slot][sl] = jnp.where(m, iv - lo, iv >> 2)
                val_s[slot][sl] = jnp.where(m, vv, 0.0)
                return _

            lax.fori_loop(0, P // LANES, remap_body, None)
            pltpu.make_async_copy(
                val_s[slot], acc.at[idx_s[slot]], ssem[slot]
            ).start(add=True)
            if p + 1 < NPIECE:
                if p >= 1:
                    pltpu.make_async_copy(
                        val_s[1 - slot], acc.at[idx_s[1 - slot]], ssem[1 - slot]
                    ).wait()
                start_loads(b, p + 1, 1 - slot)

        pltpu.make_async_copy(val0, acc.at[idx0], ssem0).wait()
        pltpu.make_async_copy(val1, acc.at[idx1], ssem1).wait()
        plsc.subcore_barrier()

        pltpu.sync_copy(
            acc.at[pl.ds(sid * NTZ, NTZ)],
            out_hbm.at[pl.ds(b * M + lo + sid * NTZ, NTZ)],
        )
        plsc.subcore_barrier()
        return _

    lax.fori_loop(0, NPASS // NC, pass_body, None)


@jax.jit
def _unpool(grad_flat, arg_flat):
    mesh = plsc.VectorSubcoreMesh(core_axis_name="c", subcore_axis_name="s")
    return pl.kernel(
        _unpool_body,
        out_type=jax.ShapeDtypeStruct((B * M,), jnp.float32),
        mesh=mesh,
        compiler_params=pltpu.CompilerParams(needs_layout_passes=False),
        scratch_types=[
            pltpu.VMEM_SHARED((CHUNK,), jnp.float32),
            pltpu.VMEM((P,), jnp.int32),
            pltpu.VMEM((P,), jnp.float32),
            pltpu.VMEM((P,), jnp.int32),
            pltpu.VMEM((P,), jnp.float32),
            pltpu.SemaphoreType.DMA,
            pltpu.SemaphoreType.DMA,
            pltpu.SemaphoreType.DMA,
            pltpu.SemaphoreType.DMA,
        ],
    )(grad_flat, arg_flat)


def kernel(grad_out, inputs, argmax, batch_size):
    del inputs, batch_size
    grad_flat = grad_out.reshape(B * N)
    arg_flat = argmax.reshape(B * N).astype(jnp.int32)
    out_flat = _unpool(grad_flat, arg_flat)
    return out_flat.reshape(B, 2 * H, 2 * W, C)


# speedup vs baseline: 2.5048x; 1.2279x over previous
import jax
import jax.numpy as jnp
from jax import lax
from jax.experimental import pallas as pl
from jax.experimental.pallas import tpu as pltpu
from jax.experimental.pallas import tpu_sc as plsc

B = 8
H = W = 112
C = 96
N = H * W * C
M = 4 * N
NC = 2
NS = 16
LANES = 16
NCHUNK = 3
CHUNK = M // NCHUNK
NPASS = B * NCHUNK
NT = N // NS
P = 6272
NPIECE = NT // P
NTZ = CHUNK // NS
ZB = 3136
NZP = NTZ // ZB
UNROLL = 8


def _unpool_body(grad_hbm, arg_hbm, out_hbm, acc,
                 idx0, val0, idx1, val1, zer_v,
                 lsem0, lsem1, ssem0, ssem1, zsem):
    cid = lax.axis_index("c")
    sid = lax.axis_index("s")

    idx_s = (idx0, idx1)
    val_s = (val0, val1)
    lsem = (lsem0, lsem1)
    ssem = (ssem0, ssem1)

    zeros16 = jnp.zeros((LANES,), jnp.float32)

    def fill_zer(i, _):
        zer_v[pl.ds(i * LANES, LANES)] = zeros16
        return _

    lax.fori_loop(0, ZB // LANES, fill_zer, None)

    def start_loads(b, p, slot):
        base = b * N + sid * NT + p * P
        pltpu.make_async_copy(
            arg_hbm.at[pl.ds(base, P)], idx_s[slot], lsem[slot]).start()
        pltpu.make_async_copy(
            grad_hbm.at[pl.ds(base, P)], val_s[slot], lsem[slot]).start()

    def wait_loads(b, p, slot):
        base = b * N + sid * NT + p * P
        pltpu.make_async_copy(
            arg_hbm.at[pl.ds(base, P)], idx_s[slot], lsem[slot]).wait()
        pltpu.make_async_copy(
            grad_hbm.at[pl.ds(base, P)], val_s[slot], lsem[slot]).wait()

    def pass_body(k, _):
        t = 2 * k + cid
        b = t // NCHUNK
        ch = t % NCHUNK
        lo = ch * CHUNK

        def zero_body(j, _):
            pltpu.make_async_copy(
                zer_v, acc.at[pl.ds(sid * NTZ + j * ZB, ZB)], zsem).start()
            return _

        lax.fori_loop(0, NZP, zero_body, None)

        def zero_wait(j, _):
            pltpu.make_async_copy(
                zer_v, acc.at[pl.ds(sid * NTZ + j * ZB, ZB)], zsem).wait()
            return _

        lax.fori_loop(0, NZP, zero_wait, None)
        plsc.subcore_barrier()

        start_loads(b, 0, 0)
        for p in range(NPIECE):
            slot = p % 2
            wait_loads(b, p, slot)

            def remap_body(i, _, slot=slot):
                for u in range(UNROLL):
                    sl = pl.ds(i * (LANES * UNROLL) + u * LANES, LANES)
                    iv = idx_s[slot][sl]
                    vv = val_s[slot][sl]
                    m = (iv >= lo) & (iv < lo + CHUNK)
                    idx_s[slot][sl] = jnp.where(m, iv - lo, iv >> 2)
                    val_s[slot][sl] = jnp.where(m, vv, 0.0)
                return _

            lax.fori_loop(0, P // (LANES * UNROLL), remap_body, None)
            pltpu.make_async_copy(
                val_s[slot], acc.at[idx_s[slot]], ssem[slot]
            ).start(add=True)
            if p + 1 < NPIECE:
                if p >= 1:
                    pltpu.make_async_copy(
                        val_s[1 - slot], acc.at[idx_s[1 - slot]], ssem[1 - slot]
                    ).wait()
                start_loads(b, p + 1, 1 - slot)

        pltpu.make_async_copy(val0, acc.at[idx0], ssem0).wait()
        pltpu.make_async_copy(val1, acc.at[idx1], ssem1).wait()
        plsc.subcore_barrier()

        pltpu.sync_copy(
            acc.at[pl.ds(sid * NTZ, NTZ)],
            out_hbm.at[pl.ds(b * M + lo + sid * NTZ, NTZ)],
        )
        return _

    lax.fori_loop(0, NPASS // NC, pass_body, None)


@jax.jit
def _unpool(grad_flat, arg_flat):
    mesh = plsc.VectorSubcoreMesh(core_axis_name="c", subcore_axis_name="s")
    return pl.kernel(
        _unpool_body,
        out_type=jax.ShapeDtypeStruct((B * M,), jnp.float32),
        mesh=mesh,
        compiler_params=pltpu.CompilerParams(needs_layout_passes=False),
        scratch_types=[
            pltpu.VMEM_SHARED((CHUNK,), jnp.float32),
            pltpu.VMEM((P,), jnp.int32),
            pltpu.VMEM((P,), jnp.float32),
            pltpu.VMEM((P,), jnp.int32),
            pltpu.VMEM((P,), jnp.float32),
            pltpu.VMEM((ZB,), jnp.float32),
            pltpu.SemaphoreType.DMA,
            pltpu.SemaphoreType.DMA,
            pltpu.SemaphoreType.DMA,
            pltpu.SemaphoreType.DMA,
            pltpu.SemaphoreType.DMA,
        ],
    )(grad_flat, arg_flat)


def kernel(grad_out, inputs, argmax, batch_size):
    del inputs, batch_size
    grad_flat = grad_out.reshape(B * N)
    arg_flat = argmax.reshape(B * N).astype(jnp.int32)
    out_flat = _unpool(grad_flat, arg_flat)
    return out_flat.reshape(B, 2 * H, 2 * W, C)
